# baseline (speedup 1.0000x reference)
import functools

import jax
import jax.numpy as jnp
from jax import lax
from jax.experimental import pallas as pl
from jax.experimental.pallas import tpu as pltpu
from jax.experimental.pallas import tpu_sc as plsc

N = 4096
D = 1024
NE = 8
ES = 7
TOPK = 2
A = N * TOPK
T = 256
NT = 40
S = NT * T
NT_PAD = 48
BT = 256
NTB = N // BT
LANE = 128

W16 = 16
EPW = A // W16
ZCH = S // W16
NW = 32
RPW = S // NW
GCH = 40
TPW = N // NW
CT = 16


def _router_kernel(x_ref, rw_ref, rb_ref, idx_ref, wts_ref):
    x = x_ref[...]
    logits = jnp.dot(x, rw_ref[...], preferred_element_type=jnp.float32)
    lane = jax.lax.broadcasted_iota(jnp.int32, (BT, LANE), 1)
    valid = lane < ES
    logits = jnp.where(valid, logits + rb_ref[0][None, :], -1e30)
    m = jnp.max(logits, axis=1, keepdims=True)
    ex = jnp.where(valid, jnp.exp(logits - m), 0.0)
    p = ex / jnp.sum(ex, axis=1, keepdims=True)
    m1 = jnp.max(p, axis=1, keepdims=True)
    a1 = jnp.min(jnp.where(p == m1, lane, LANE), axis=1, keepdims=True)
    p2 = jnp.where(lane == a1, -1.0, p)
    m2 = jnp.max(p2, axis=1, keepdims=True)
    a2 = jnp.min(jnp.where(p2 == m2, lane, LANE), axis=1, keepdims=True)
    den = m1 + m2 + 1e-6
    idx_ref[:, 0:1] = a1
    idx_ref[:, 1:2] = a2
    wts_ref[:, 0:1] = m1 / den
    wts_ref[:, 1:2] = m2 / den


def _k1(x, rw, rb):
    return pl.pallas_call(
        _router_kernel,
        grid=(NTB,),
        in_specs=[
            pl.BlockSpec((BT, D), lambda t: (t, 0)),
            pl.BlockSpec((D, LANE), lambda t: (0, 0)),
            pl.BlockSpec((1, LANE), lambda t: (0, 0)),
        ],
        out_specs=[
            pl.BlockSpec((BT, TOPK), lambda t: (t, 0)),
            pl.BlockSpec((BT, TOPK), lambda t: (t, 0)),
        ],
        out_shape=[
            jax.ShapeDtypeStruct((N, TOPK), jnp.int32),
            jax.ShapeDtypeStruct((N, TOPK), jnp.float32),
        ],
        compiler_params=pltpu.CompilerParams(dimension_semantics=("arbitrary",)),
    )(x, rw, rb)


def _shared_kernel(x_ref, w_ref, b_ref, ysh_ref):
    ysh_ref[...] = (
        jnp.dot(x_ref[...], w_ref[0], preferred_element_type=jnp.float32) + b_ref[0]
    )


def _k1b(x, expert_w, eb3):
    return pl.pallas_call(
        _shared_kernel,
        grid=(NTB,),
        in_specs=[
            pl.BlockSpec((BT, D), lambda t: (t, 0)),
            pl.BlockSpec((1, D, D), lambda t: (0, 0, 0)),
            pl.BlockSpec((1, 1, D), lambda t: (0, 0, 0)),
        ],
        out_specs=pl.BlockSpec((BT, D), lambda t: (t, 0)),
        out_shape=jax.ShapeDtypeStruct((N, D), jnp.float32),
        compiler_params=pltpu.CompilerParams(dimension_semantics=("arbitrary",)),
    )(x, expert_w, eb3)


_MESH1 = plsc.VectorSubcoreMesh(
    core_axis_name="c", subcore_axis_name="s", num_cores=1, num_subcores=16)


@functools.partial(
    pl.kernel,
    out_type=[
        jax.ShapeDtypeStruct((S,), jnp.int32),
        jax.ShapeDtypeStruct((A,), jnp.int32),
        jax.ShapeDtypeStruct((NT_PAD,), jnp.int32),
    ],
    mesh=_MESH1,
    scratch_types=[
        pltpu.VMEM((EPW,), jnp.int32),
        pltpu.VMEM((4, 128), jnp.int32),
        pltpu.VMEM((4, 128), jnp.int32),
        pltpu.VMEM((W16 * 16,), jnp.int32),
        pltpu.VMEM((ZCH,), jnp.int32),
        pltpu.VMEM((NT_PAD,), jnp.int32),
        pltpu.VMEM_SHARED((W16 * 16,), jnp.int32),
        pltpu.VMEM_SHARED((S,), jnp.int32),
        pltpu.SemaphoreType.DMA,
    ],
    compiler_params=pltpu.CompilerParams(needs_layout_passes=False),
)
def _sort_kernel(idxflat_hbm, sorted_hbm, pos_hbm, te_hbm,
                 idx_buf, pos_buf, tok_buf, cnt_v, zero_v, te_v,
                 counts_sh, sorted_sh, sem):
    w = lax.axis_index("s")
    base_a = w * EPW
    iota = lax.iota(jnp.int32, 16)
    zeros16 = jnp.zeros((16,), jnp.int32)

    pltpu.sync_copy(idxflat_hbm.at[pl.ds(base_a, EPW)], idx_buf)

    def _cbody(j, cnt):
        v = idx_buf[pl.ds(j * 16, 16)]
        for e in range(ES):
            c = plsc.all_reduce_population_count(v == e)
            cnt = cnt + jnp.where(iota == e, c, 0)
        return cnt

    cnt = lax.fori_loop(0, EPW // 16, _cbody, zeros16)
    cnt_v[pl.ds(0, 16)] = cnt
    pltpu.sync_copy(cnt_v.at[pl.ds(0, 16)], counts_sh.at[pl.ds(w * 16, 16)])

    def _zbody(j, _):
        zero_v[pl.ds(j * 16, 16)] = zeros16
        return 0

    lax.fori_loop(0, ZCH // 16, _zbody, 0)
    pltpu.sync_copy(zero_v, sorted_sh.at[pl.ds(w * ZCH, ZCH)])
    plsc.subcore_barrier()

    pltpu.sync_copy(counts_sh, cnt_v)
    acc = zeros16
    mystart = zeros16
    for wp in range(W16):
        mystart = jnp.where(w == wp, acc, mystart)
        acc = acc + cnt_v[pl.ds(wp * 16, 16)]
    g = acc
    p_pad = ((g + (T - 1)) >> 8) << 8
    csum = plsc.cumsum(p_pad)
    base = csum - p_pad
    start_vec = base + mystart
    cumtiles = csum >> 8

    @pl.when(w == 0)
    def _te():
        for c in range(NT_PAD // 16):
            tvec = iota + c * 16
            te = jnp.zeros((16,), jnp.int32)
            for e in range(ES):
                ct_e = jnp.sum(jnp.where(iota == e, cumtiles, 0))
                te = te + jnp.where(ct_e <= tvec, 1, 0)
            te_v[pl.ds(c * 16, 16)] = jnp.minimum(te, ES - 1)
        pltpu.sync_copy(te_v, te_hbm)

    rc = zeros16
    for j in range(EPW // 16):
        v = idx_buf[pl.ds(j * 16, 16)]
        tok = (base_a + j * 16 + iota) >> 1
        pos = zeros16
        for e in range(ES):
            m = v == e
            cs = plsc.cumsum(jnp.where(m, 1, 0))
            start_e = jnp.sum(jnp.where(iota == e, start_vec + rc, 0))
            pos = jnp.where(m, start_e + cs - 1, pos)
            rc = rc + jnp.where(iota == e, plsc.all_reduce_population_count(m), 0)
        pos_buf[j // 8, pl.ds((j % 8) * 16, 16)] = pos
        tok_buf[j // 8, pl.ds((j % 8) * 16, 16)] = tok
    for c in range(4):
        pltpu.async_copy(tok_buf.at[c], sorted_sh.at[pos_buf.at[c]], sem).wait()
        pltpu.sync_copy(pos_buf.at[c], pos_hbm.at[pl.ds(base_a + c * 128, 128)])
    plsc.subcore_barrier()

    pltpu.sync_copy(sorted_sh.at[pl.ds(w * ZCH, ZCH)], zero_v)
    pltpu.sync_copy(zero_v, sorted_hbm.at[pl.ds(w * ZCH, ZCH)])


_MESH2 = plsc.VectorSubcoreMesh(
    core_axis_name="c", subcore_axis_name="s", num_cores=2, num_subcores=16)


_NCH = RPW // GCH


@functools.partial(
    pl.kernel,
    out_type=jax.ShapeDtypeStruct((S, D // 2), jnp.int32),
    mesh=_MESH2,
    scratch_types=[
        pltpu.VMEM((RPW,), jnp.int32),
        pltpu.VMEM((GCH, D // 2), jnp.int32),
        pltpu.VMEM((GCH, D // 2), jnp.int32),
        pltpu.SemaphoreType.DMA,
        pltpu.SemaphoreType.DMA,
    ],
    compiler_params=pltpu.CompilerParams(needs_layout_passes=False),
)
def _gather_kernel(x_hbm, tok_hbm, xg_hbm, idx_v, rows_a, rows_b, sem_a, sem_b):
    wid = lax.axis_index("s") * 2 + lax.axis_index("c")
    base = wid * RPW
    bufs = (rows_a, rows_b)
    sems = (sem_a, sem_b)
    pltpu.sync_copy(tok_hbm.at[pl.ds(base, RPW)], idx_v)
    pend = [None, None]
    pend[0] = pltpu.async_copy(x_hbm.at[idx_v.at[pl.ds(0, GCH)]], rows_a, sem_a)
    for c in range(_NCH):
        pend[c % 2].wait()
        if c + 1 < _NCH:
            pend[(c + 1) % 2] = pltpu.async_copy(
                x_hbm.at[idx_v.at[pl.ds((c + 1) * GCH, GCH)]],
                bufs[(c + 1) % 2], sems[(c + 1) % 2])
        pltpu.sync_copy(bufs[c % 2], xg_hbm.at[pl.ds(base + c * GCH, GCH)])


def _gmm_kernel(te_ref, xg_ref, w_ref, b_ref, out_ref):
    out_ref[...] = (
        jnp.dot(xg_ref[...], w_ref[0], preferred_element_type=jnp.float32) + b_ref[0]
    )


def _k4(te_arr, xg, expert_w, eb3):
    grid_spec = pltpu.PrefetchScalarGridSpec(
        num_scalar_prefetch=1,
        grid=(NT,),
        in_specs=[
            pl.BlockSpec((T, D), lambda t, te: (t, 0)),
            pl.BlockSpec((1, D, D), lambda t, te: (te[t] + 1, 0, 0)),
            pl.BlockSpec((1, 1, D), lambda t, te: (te[t] + 1, 0, 0)),
        ],
        out_specs=pl.BlockSpec((T, D), lambda t, te: (t, 0)),
    )
    return pl.pallas_call(
        _gmm_kernel,
        grid_spec=grid_spec,
        out_shape=jax.ShapeDtypeStruct((S, D), jnp.float32),
        compiler_params=pltpu.CompilerParams(dimension_semantics=("arbitrary",)),
    )(te_arr, xg, expert_w.astype(jnp.bfloat16), eb3)


_CCH = TPW // CT


@functools.partial(
    pl.kernel,
    out_type=jax.ShapeDtypeStruct((N, D), jnp.float32),
    mesh=_MESH2,
    scratch_types=[
        pltpu.VMEM((2 * TPW,), jnp.int32),
        pltpu.VMEM((2 * TPW,), jnp.float32),
        pltpu.VMEM((2 * CT, D), jnp.float32),
        pltpu.VMEM((2 * CT, D), jnp.float32),
        pltpu.VMEM((CT, D), jnp.float32),
        pltpu.VMEM((CT, D), jnp.float32),
        pltpu.SemaphoreType.DMA,
        pltpu.SemaphoreType.DMA,
    ],
)
def _combine_kernel(og_hbm, pos_hbm, w_hbm, ysh_hbm, y_hbm,
                    pos_v, w_v, rows_a, rows_b, ysh_v, out_v, sem_a, sem_b):
    wid = lax.axis_index("s") * 2 + lax.axis_index("c")
    tbase = wid * TPW
    bufs = (rows_a, rows_b)
    sems = (sem_a, sem_b)
    pltpu.sync_copy(pos_hbm.at[pl.ds(2 * tbase, 2 * TPW)], pos_v)
    pltpu.sync_copy(w_hbm.at[pl.ds(2 * tbase, 2 * TPW)], w_v)
    pend = [None, None]
    pend[0] = pltpu.async_copy(og_hbm.at[pos_v.at[pl.ds(0, 2 * CT)]], rows_a, sem_a)
    for c in range(_CCH):
        t0 = tbase + c * CT
        pltpu.sync_copy(ysh_hbm.at[pl.ds(t0, CT)], ysh_v)
        pend[c % 2].wait()
        if c + 1 < _CCH:
            pend[(c + 1) % 2] = pltpu.async_copy(
                og_hbm.at[pos_v.at[pl.ds((c + 1) * 2 * CT, 2 * CT)]],
                bufs[(c + 1) % 2], sems[(c + 1) % 2])
        rows_v = bufs[c % 2]
        wa = w_v[pl.ds(c * 2 * CT, 16)]
        wb = w_v[pl.ds(c * 2 * CT + 16, 16)]
        for i in range(CT):
            src = wa if i < 8 else wb
            w0 = src[(2 * i) % 16]
            w1 = src[(2 * i + 1) % 16]

            def _dbody(dc, _, rows_v=rows_v, i=i, w0=w0, w1=w1):
                sl = pl.ds(dc * 16, 16)
                out_v[i, sl] = ysh_v[i, sl] + w0 * rows_v[2 * i, sl] + w1 * rows_v[2 * i + 1, sl]
                return 0

            lax.fori_loop(0, D // 16, _dbody, 0)
        pltpu.sync_copy(out_v, y_hbm.at[pl.ds(t0, CT)])


def kernel(x, expert_w, expert_b, router_w, router_b):
    rw = jnp.zeros((D, LANE), jnp.float32).at[:, :ES].set(router_w)
    rb = jnp.zeros((1, LANE), jnp.float32).at[0, :ES].set(router_b)
    eb3 = expert_b.reshape(NE, 1, D)
    tidx, tw = _k1(x, rw, rb)
    ysh = _k1b(x, expert_w, eb3)
    idx_flat = tidx.reshape(A)
    w_flat = tw.reshape(A)
    sorted_tok, pos_flat, te_arr = _sort_kernel(idx_flat)
    x32 = lax.bitcast_convert_type(
        x.astype(jnp.bfloat16).reshape(N, D // 2, 2), jnp.int32)
    xg32 = _gather_kernel(x32, sorted_tok)
    xg = lax.bitcast_convert_type(xg32, jnp.bfloat16).reshape(S, D)
    og = _k4(te_arr, xg, expert_w, eb3)
    return _combine_kernel(og, pos_flat, w_flat, ysh)


# speedup vs baseline: 2.0243x; 2.0243x over previous
import functools

import jax
import jax.numpy as jnp
from jax import lax
from jax.experimental import pallas as pl
from jax.experimental.pallas import tpu as pltpu
from jax.experimental.pallas import tpu_sc as plsc

N = 4096
D = 1024
NE = 8
ES = 7
TOPK = 2
A = N * TOPK
T = 128
TSH = 7
NT = 72
S = NT * T
NT_PAD = 80
BT = 256
NTB = N // BT
LANE = 128

W16 = 16
EPW = A // W16
ZCH = S // W16
NW = 32
RPW = S // NW
GCH = 24
TPW = N // NW
CT = 16


def _router_kernel(x_ref, rw_ref, rb_ref, idx_ref, wts_ref):
    x = x_ref[...]
    logits = jnp.dot(x, rw_ref[...], preferred_element_type=jnp.float32)
    lane = jax.lax.broadcasted_iota(jnp.int32, (BT, LANE), 1)
    valid = lane < ES
    logits = jnp.where(valid, logits + rb_ref[0][None, :], -1e30)
    m = jnp.max(logits, axis=1, keepdims=True)
    ex = jnp.where(valid, jnp.exp(logits - m), 0.0)
    p = ex / jnp.sum(ex, axis=1, keepdims=True)
    m1 = jnp.max(p, axis=1, keepdims=True)
    a1 = jnp.min(jnp.where(p == m1, lane, LANE), axis=1, keepdims=True)
    p2 = jnp.where(lane == a1, -1.0, p)
    m2 = jnp.max(p2, axis=1, keepdims=True)
    a2 = jnp.min(jnp.where(p2 == m2, lane, LANE), axis=1, keepdims=True)
    den = m1 + m2 + 1e-6
    idx_ref[:, 0:1] = a1
    idx_ref[:, 1:2] = a2
    wts_ref[:, 0:1] = m1 / den
    wts_ref[:, 1:2] = m2 / den


def _k1(x, rw, rb):
    return pl.pallas_call(
        _router_kernel,
        grid=(NTB,),
        in_specs=[
            pl.BlockSpec((BT, D), lambda t: (t, 0)),
            pl.BlockSpec((D, LANE), lambda t: (0, 0)),
            pl.BlockSpec((1, LANE), lambda t: (0, 0)),
        ],
        out_specs=[
            pl.BlockSpec((BT, TOPK), lambda t: (t, 0)),
            pl.BlockSpec((BT, TOPK), lambda t: (t, 0)),
        ],
        out_shape=[
            jax.ShapeDtypeStruct((N, TOPK), jnp.int32),
            jax.ShapeDtypeStruct((N, TOPK), jnp.float32),
        ],
        compiler_params=pltpu.CompilerParams(dimension_semantics=("arbitrary",)),
    )(x, rw, rb)


def _shared_kernel(x_ref, w_ref, b_ref, ysh_ref):
    ysh_ref[...] = (
        jnp.dot(x_ref[...], w_ref[0], preferred_element_type=jnp.float32) + b_ref[0]
    )


def _k1b(x, expert_w, eb3):
    return pl.pallas_call(
        _shared_kernel,
        grid=(NTB,),
        in_specs=[
            pl.BlockSpec((BT, D), lambda t: (t, 0)),
            pl.BlockSpec((1, D, D), lambda t: (0, 0, 0)),
            pl.BlockSpec((1, 1, D), lambda t: (0, 0, 0)),
        ],
        out_specs=pl.BlockSpec((BT, D), lambda t: (t, 0)),
        out_shape=jax.ShapeDtypeStruct((N, D), jnp.float32),
        compiler_params=pltpu.CompilerParams(dimension_semantics=("arbitrary",)),
    )(x, expert_w, eb3)


_MESH1 = plsc.VectorSubcoreMesh(
    core_axis_name="c", subcore_axis_name="s", num_cores=1, num_subcores=16)


@functools.partial(
    pl.kernel,
    out_type=[
        jax.ShapeDtypeStruct((S,), jnp.int32),
        jax.ShapeDtypeStruct((A,), jnp.int32),
        jax.ShapeDtypeStruct((NT_PAD,), jnp.int32),
    ],
    mesh=_MESH1,
    scratch_types=[
        pltpu.VMEM((EPW,), jnp.int32),
        pltpu.VMEM((4, 128), jnp.int32),
        pltpu.VMEM((4, 128), jnp.int32),
        pltpu.VMEM((W16 * 16,), jnp.int32),
        pltpu.VMEM((ZCH,), jnp.int32),
        pltpu.VMEM((NT_PAD,), jnp.int32),
        pltpu.VMEM_SHARED((W16 * 16,), jnp.int32),
        pltpu.VMEM_SHARED((S,), jnp.int32),
        pltpu.SemaphoreType.DMA,
    ],
    compiler_params=pltpu.CompilerParams(needs_layout_passes=False),
)
def _sort_kernel(idxflat_hbm, sorted_hbm, pos_hbm, te_hbm,
                 idx_buf, pos_buf, tok_buf, cnt_v, zero_v, te_v,
                 counts_sh, sorted_sh, sem):
    w = lax.axis_index("s")
    base_a = w * EPW
    iota = lax.iota(jnp.int32, 16)
    zeros16 = jnp.zeros((16,), jnp.int32)

    pltpu.sync_copy(idxflat_hbm.at[pl.ds(base_a, EPW)], idx_buf)

    def _cbody(j, cnt):
        v = idx_buf[pl.ds(j * 16, 16)]
        for e in range(ES):
            c = plsc.all_reduce_population_count(v == e)
            cnt = cnt + jnp.where(iota == e, c, 0)
        return cnt

    cnt = lax.fori_loop(0, EPW // 16, _cbody, zeros16)
    cnt_v[pl.ds(0, 16)] = cnt
    pltpu.sync_copy(cnt_v.at[pl.ds(0, 16)], counts_sh.at[pl.ds(w * 16, 16)])

    def _zbody(j, _):
        zero_v[pl.ds(j * 16, 16)] = zeros16
        return 0

    lax.fori_loop(0, ZCH // 16, _zbody, 0)
    pltpu.sync_copy(zero_v, sorted_sh.at[pl.ds(w * ZCH, ZCH)])
    plsc.subcore_barrier()

    pltpu.sync_copy(counts_sh, cnt_v)
    acc = zeros16
    mystart = zeros16
    for wp in range(W16):
        mystart = jnp.where(w == wp, acc, mystart)
        acc = acc + cnt_v[pl.ds(wp * 16, 16)]
    g = acc
    p_pad = ((g + (T - 1)) >> TSH) << TSH
    csum = plsc.cumsum(p_pad)
    base = csum - p_pad
    start_vec = base + mystart
    cumtiles = csum >> TSH

    @pl.when(w == 0)
    def _te():
        for c in range(NT_PAD // 16):
            tvec = iota + c * 16
            te = jnp.zeros((16,), jnp.int32)
            for e in range(ES):
                ct_e = jnp.sum(jnp.where(iota == e, cumtiles, 0))
                te = te + jnp.where(ct_e <= tvec, 1, 0)
            te_v[pl.ds(c * 16, 16)] = jnp.minimum(te, ES - 1)
        pltpu.sync_copy(te_v, te_hbm)

    rc = zeros16
    for j in range(EPW // 16):
        v = idx_buf[pl.ds(j * 16, 16)]
        tok = (base_a + j * 16 + iota) >> 1
        pos = zeros16
        for e in range(ES):
            m = v == e
            cs = plsc.cumsum(jnp.where(m, 1, 0))
            start_e = jnp.sum(jnp.where(iota == e, start_vec + rc, 0))
            pos = jnp.where(m, start_e + cs - 1, pos)
            rc = rc + jnp.where(iota == e, plsc.all_reduce_population_count(m), 0)
        pos_buf[j // 8, pl.ds((j % 8) * 16, 16)] = pos
        tok_buf[j // 8, pl.ds((j % 8) * 16, 16)] = tok
    for c in range(4):
        pltpu.async_copy(tok_buf.at[c], sorted_sh.at[pos_buf.at[c]], sem).wait()
        pltpu.sync_copy(pos_buf.at[c], pos_hbm.at[pl.ds(base_a + c * 128, 128)])
    plsc.subcore_barrier()

    pltpu.sync_copy(sorted_sh.at[pl.ds(w * ZCH, ZCH)], zero_v)
    pltpu.sync_copy(zero_v, sorted_hbm.at[pl.ds(w * ZCH, ZCH)])


_MESH2 = plsc.VectorSubcoreMesh(
    core_axis_name="c", subcore_axis_name="s", num_cores=2, num_subcores=16)


_NCH = RPW // GCH
_NBUF = 4


@functools.partial(
    pl.kernel,
    out_type=jax.ShapeDtypeStruct((S, D), jnp.float32),
    mesh=_MESH2,
    scratch_types=(
        [pltpu.VMEM((RPW,), jnp.int32)]
        + [pltpu.VMEM((GCH, D), jnp.float32) for _ in range(_NBUF)]
        + [pltpu.SemaphoreType.DMA for _ in range(2 * _NBUF)]
    ),
    compiler_params=pltpu.CompilerParams(needs_layout_passes=False),
)
def _gather_kernel(x_hbm, tok_hbm, xg_hbm, idx_v, *bufs_sems):
    bufs = bufs_sems[:_NBUF]
    gsems = bufs_sems[_NBUF:2 * _NBUF]
    wsems = bufs_sems[2 * _NBUF:]
    wid = lax.axis_index("s") * 2 + lax.axis_index("c")
    base = wid * RPW
    pltpu.sync_copy(tok_hbm.at[pl.ds(base, RPW)], idx_v)
    pend_g = [None] * _NBUF
    pend_w = [None] * _NBUF
    for c in range(min(_NBUF - 1, _NCH)):
        pend_g[c] = pltpu.async_copy(
            x_hbm.at[idx_v.at[pl.ds(c * GCH, GCH)]], bufs[c], gsems[c])
    for c in range(_NCH):
        b = c % _NBUF
        pend_g[b].wait()
        pend_w[b] = pltpu.async_copy(
            bufs[b], xg_hbm.at[pl.ds(base + c * GCH, GCH)], wsems[b])
        nxt = c + _NBUF - 1
        if nxt < _NCH:
            nb = nxt % _NBUF
            if pend_w[nb] is not None:
                pend_w[nb].wait()
            pend_g[nb] = pltpu.async_copy(
                x_hbm.at[idx_v.at[pl.ds(nxt * GCH, GCH)]], bufs[nb], gsems[nb])
    for c in range(max(0, _NCH - _NBUF), _NCH):
        pend_w[c % _NBUF].wait()


def _gmm_kernel(te_ref, xg_ref, w_ref, b_ref, out_ref):
    out_ref[...] = (
        jnp.dot(xg_ref[...].astype(jnp.bfloat16), w_ref[0],
                preferred_element_type=jnp.float32) + b_ref[0]
    )


def _k4(te_arr, xg, expert_w, eb3):
    grid_spec = pltpu.PrefetchScalarGridSpec(
        num_scalar_prefetch=1,
        grid=(NT,),
        in_specs=[
            pl.BlockSpec((T, D), lambda t, te: (t, 0)),
            pl.BlockSpec((1, D, D), lambda t, te: (te[t] + 1, 0, 0)),
            pl.BlockSpec((1, 1, D), lambda t, te: (te[t] + 1, 0, 0)),
        ],
        out_specs=pl.BlockSpec((T, D), lambda t, te: (t, 0)),
    )
    return pl.pallas_call(
        _gmm_kernel,
        grid_spec=grid_spec,
        out_shape=jax.ShapeDtypeStruct((S, D), jnp.float32),
        compiler_params=pltpu.CompilerParams(dimension_semantics=("arbitrary",)),
    )(te_arr, xg, expert_w.astype(jnp.bfloat16), eb3)


_CCH = TPW // CT


@functools.partial(
    pl.kernel,
    out_type=jax.ShapeDtypeStruct((N, D), jnp.float32),
    mesh=_MESH2,
    scratch_types=[
        pltpu.VMEM((2 * TPW,), jnp.int32),
        pltpu.VMEM((2 * TPW,), jnp.float32),
        pltpu.VMEM((2 * CT, D), jnp.float32),
        pltpu.VMEM((2 * CT, D), jnp.float32),
        pltpu.VMEM((CT, D), jnp.float32),
        pltpu.VMEM((CT, D), jnp.float32),
        pltpu.SemaphoreType.DMA,
        pltpu.SemaphoreType.DMA,
    ],
)
def _combine_kernel(og_hbm, pos_hbm, w_hbm, ysh_hbm, y_hbm,
                    pos_v, w_v, rows_a, rows_b, ysh_v, out_v, sem_a, sem_b):
    wid = lax.axis_index("s") * 2 + lax.axis_index("c")
    tbase = wid * TPW
    bufs = (rows_a, rows_b)
    sems = (sem_a, sem_b)
    pltpu.sync_copy(pos_hbm.at[pl.ds(2 * tbase, 2 * TPW)], pos_v)
    pltpu.sync_copy(w_hbm.at[pl.ds(2 * tbase, 2 * TPW)], w_v)
    pend = [None, None]
    pend[0] = pltpu.async_copy(og_hbm.at[pos_v.at[pl.ds(0, 2 * CT)]], rows_a, sem_a)
    for c in range(_CCH):
        t0 = tbase + c * CT
        pltpu.sync_copy(ysh_hbm.at[pl.ds(t0, CT)], ysh_v)
        pend[c % 2].wait()
        if c + 1 < _CCH:
            pend[(c + 1) % 2] = pltpu.async_copy(
                og_hbm.at[pos_v.at[pl.ds((c + 1) * 2 * CT, 2 * CT)]],
                bufs[(c + 1) % 2], sems[(c + 1) % 2])
        rows_v = bufs[c % 2]
        wa = w_v[pl.ds(c * 2 * CT, 16)]
        wb = w_v[pl.ds(c * 2 * CT + 16, 16)]
        for i in range(CT):
            src = wa if i < 8 else wb
            w0 = src[(2 * i) % 16]
            w1 = src[(2 * i + 1) % 16]

            def _dbody(dc, _, rows_v=rows_v, i=i, w0=w0, w1=w1):
                sl = pl.ds(dc * 16, 16)
                out_v[i, sl] = ysh_v[i, sl] + w0 * rows_v[2 * i, sl] + w1 * rows_v[2 * i + 1, sl]
                return 0

            lax.fori_loop(0, D // 16, _dbody, 0)
        pltpu.sync_copy(out_v, y_hbm.at[pl.ds(t0, CT)])


def kernel(x, expert_w, expert_b, router_w, router_b):
    rw = jnp.zeros((D, LANE), jnp.float32).at[:, :ES].set(router_w)
    rb = jnp.zeros((1, LANE), jnp.float32).at[0, :ES].set(router_b)
    eb3 = expert_b.reshape(NE, 1, D)
    tidx, tw = _k1(x, rw, rb)
    ysh = _k1b(x, expert_w, eb3)
    idx_flat = tidx.reshape(A)
    w_flat = tw.reshape(A)
    sorted_tok, pos_flat, te_arr = _sort_kernel(idx_flat)
    xg = _gather_kernel(x, sorted_tok)
    og = _k4(te_arr, xg, expert_w, eb3)
    return _combine_kernel(og, pos_flat, w_flat, ysh)
